# Initial kernel scaffold; baseline (speedup 1.0000x reference)
import jax
import jax.numpy as jnp
from jax.experimental import pallas as pl


def kernel(x, edge_index, W1, b1, W2, b2, Wl, bl):
    raise NotImplementedError("write your pallas kernel here")


# speedup vs baseline: 69.9681x; 69.9681x over previous
import functools

import jax
import jax.numpy as jnp
from jax import lax
from jax.experimental import pallas as pl
from jax.experimental.pallas import tpu as pltpu
from jax.experimental.pallas import tpu_sc as plsc

N = 50000
E = 3200000
NCORE = 2
NSUB = 16
NW = NCORE * NSUB
CHUNK = 128
BLKCH = 56
NBLK = 14
CPW = BLKCH * NBLK
E_PAD = NW * CPW * CHUNK
GRP = 7
N_PAD = 50048
STRIPE = N_PAD // NSUB
TRASH = N

BLK = 2000
NB_TC = N // BLK

_mesh = plsc.VectorSubcoreMesh(core_axis_name="c", subcore_axis_name="s")
_sc_params = pltpu.CompilerParams(use_tc_tiling_on_sc=False)


def _zero_stripe(z_hbm, acc, s):
    sl = pl.ds(s * STRIPE, STRIPE)
    pltpu.sync_copy(z_hbm.at[sl], acc.at[sl])


def _writeback_stripe(acc, out_hbm, c, s):
    sl = pl.ds(s * STRIPE, STRIPE)
    pltpu.sync_copy(acc.at[sl], out_hbm.at[c, sl])


def _scatter_const_row(dst3d_hbm, ones_hbm, z_hbm, out_hbm, dstb, ones_v, acc,
                       sem_s, sem_i):
    c = lax.axis_index("c")
    s = lax.axis_index("s")
    w = 2 * s + c
    _zero_stripe(z_hbm, acc, s)
    pltpu.async_copy(ones_hbm, ones_v, sem_i).wait()
    plsc.subcore_barrier()

    @pl.loop(0, NBLK)
    def _blk(b):
        pltpu.async_copy(dst3d_hbm.at[w, pl.ds(b * BLKCH, BLKCH)], dstb,
                         sem_i).wait()

        @pl.loop(0, BLKCH // GRP)
        def _grp(gg):
            for p in range(GRP):
                j = gg * GRP + p
                pltpu.async_copy(ones_v, acc.at[dstb.at[j]], sem_s.at[p],
                                 add=True)
            for p in range(GRP):
                j = gg * GRP + p
                pltpu.make_async_copy(ones_v, acc.at[dstb.at[j]],
                                      sem_s.at[p]).wait()

    plsc.subcore_barrier()
    _writeback_stripe(acc, out_hbm, c, s)


def _make_agg_kernel(width, per_core_features):
    def inner(src_hbm, dst3d_hbm, table_hbm, acc, srcb, dstb, rows,
              sem_g, sem_s, sem_i, w, qq):
        @pl.loop(0, NBLK)
        def _blk(b):
            pltpu.async_copy(src_hbm.at[w, pl.ds(b * BLKCH, BLKCH)], srcb,
                             sem_i).wait()
            pltpu.async_copy(dst3d_hbm.at[w, pl.ds(b * BLKCH, BLKCH)],
                             dstb, sem_i).wait()
            if per_core_features:
                @pl.loop(0, BLKCH)
                def _fix(r):
                    for k in range(CHUNK // 16):
                        sl = pl.ds(k * 16, 16)
                        srcb[r, sl] = srcb[r, sl] + qq

            @pl.loop(0, BLKCH // GRP)
            def _grp(gg):
                for p in range(GRP):
                    j = gg * GRP + p
                    pltpu.async_copy(table_hbm.at[srcb.at[j]], rows.at[p],
                                     sem_g.at[p])
                for p in range(GRP):
                    j = gg * GRP + p
                    pltpu.make_async_copy(table_hbm.at[srcb.at[j]],
                                          rows.at[p], sem_g.at[p]).wait()
                    pltpu.async_copy(rows.at[p], acc.at[dstb.at[j]],
                                     sem_s.at[p], add=True)
                for p in range(GRP):
                    j = gg * GRP + p
                    pltpu.make_async_copy(rows.at[p], acc.at[dstb.at[j]],
                                          sem_s.at[p]).wait()

    def body(src_hbm, dst3d_hbm, table_hbm, z_hbm, out_hbm,
             srcb, dstb, rows, acc, sem_g, sem_s, sem_i):
        c = lax.axis_index("c")
        s = lax.axis_index("s")

        if per_core_features:
            for q in range(2):
                qq = 2 * c + q
                _zero_stripe(z_hbm, acc, s)
                plsc.subcore_barrier()
                for rr in range(2):
                    inner(src_hbm, dst3d_hbm, table_hbm, acc, srcb, dstb,
                          rows, sem_g, sem_s, sem_i, 2 * s + rr, qq)
                plsc.subcore_barrier()
                sl = pl.ds(s * STRIPE, STRIPE)
                pltpu.sync_copy(acc.at[sl], out_hbm.at[qq, sl])
                plsc.subcore_barrier()
        else:
            _zero_stripe(z_hbm, acc, s)
            plsc.subcore_barrier()
            inner(src_hbm, dst3d_hbm, table_hbm, acc, srcb, dstb, rows,
                  sem_g, sem_s, sem_i, 2 * s + c, 0)
            plsc.subcore_barrier()
            _writeback_stripe(acc, out_hbm, c, s)

    return body


def _sc_deg(dst3d, ones_rows, z16):
    kern = functools.partial(
        pl.kernel,
        out_type=jax.ShapeDtypeStruct((NCORE, N_PAD, 16), jnp.float32),
        mesh=_mesh,
        compiler_params=_sc_params,
        scratch_types=[
            pltpu.VMEM((BLKCH, CHUNK), jnp.int32),
            pltpu.VMEM((CHUNK, 16), jnp.float32),
            pltpu.VMEM_SHARED((N_PAD, 16), jnp.float32),
            pltpu.SemaphoreType.DMA((GRP,)),
            pltpu.SemaphoreType.DMA,
        ],
    )(_scatter_const_row)
    return kern(dst3d, ones_rows, z16)


def _sc_agg(src_idx, dst3d, table, zeros, per_core_features):
    body = _make_agg_kernel(16, per_core_features)
    nout = 4 if per_core_features else NCORE
    kern = functools.partial(
        pl.kernel,
        out_type=jax.ShapeDtypeStruct((nout, N_PAD, 16), jnp.float32),
        mesh=_mesh,
        compiler_params=_sc_params,
        scratch_types=[
            pltpu.VMEM((BLKCH, CHUNK), jnp.int32),
            pltpu.VMEM((BLKCH, CHUNK), jnp.int32),
            pltpu.VMEM((GRP, CHUNK, 16), jnp.float32),
            pltpu.VMEM_SHARED((N_PAD, 16), jnp.float32),
            pltpu.SemaphoreType.DMA((GRP,)),
            pltpu.SemaphoreType.DMA((GRP,)),
            pltpu.SemaphoreType.DMA,
        ],
    )(body)
    return kern(src_idx, dst3d, table, zeros)


def _tc1_body(parts_ref, xpad_ref, dinv_ref, g_ref):
    deg = parts_ref[0, :, 0:1] + parts_ref[1, :, 0:1] + 1.0
    dv = lax.rsqrt(deg)
    dinv_ref[...] = dv
    g_ref[...] = xpad_ref[...] * dv


def _tc1(parts, xpad):
    return pl.pallas_call(
        _tc1_body,
        grid=(NB_TC,),
        in_specs=[
            pl.BlockSpec((NCORE, BLK, 16), lambda i: (0, i, 0)),
            pl.BlockSpec((BLK, 16), lambda i: (i, 0)),
        ],
        out_specs=[
            pl.BlockSpec((BLK, 1), lambda i: (i, 0)),
            pl.BlockSpec((BLK, 16), lambda i: (i, 0)),
        ],
        out_shape=[
            jax.ShapeDtypeStruct((N, 1), jnp.float32),
            jax.ShapeDtypeStruct((N, 16), jnp.float32),
        ],
    )(parts, xpad)


def _tc2_body(agg_ref, g_ref, dinv_ref, w1_ref, b1_ref, u_ref):
    a = agg_ref[0] + agg_ref[1] + g_ref[...]
    dv = dinv_ref[...]
    y = dv * a[:, 0:2]
    h = (y[:, 0:1] * w1_ref[0:1, :] + y[:, 1:2] * w1_ref[1:2, :]
         + b1_ref[...])
    h = jnp.maximum(h, 0.0)
    u_ref[...] = dv * h


def _tc2(agg1, g, dinv, W1, b1):
    return pl.pallas_call(
        _tc2_body,
        grid=(NB_TC,),
        in_specs=[
            pl.BlockSpec((NCORE, BLK, 16), lambda i: (0, i, 0)),
            pl.BlockSpec((BLK, 16), lambda i: (i, 0)),
            pl.BlockSpec((BLK, 1), lambda i: (i, 0)),
            pl.BlockSpec((2, 64), lambda i: (0, 0)),
            pl.BlockSpec((1, 64), lambda i: (0, 0)),
        ],
        out_specs=pl.BlockSpec((BLK, 64), lambda i: (i, 0)),
        out_shape=jax.ShapeDtypeStruct((N, 64), jnp.float32),
    )(agg1, g, dinv, W1, b1)


def _tc3_body(agg_ref, u_ref, dinv_ref, w2_ref, b2_ref, wlt_ref, bl_ref,
              out_ref, acc_ref):
    i = pl.program_id(0)
    dv = dinv_ref[...]
    agg = jnp.concatenate([agg_ref[0], agg_ref[1], agg_ref[2], agg_ref[3]],
                          axis=1)
    v = dv * (agg + u_ref[...])
    h2 = jnp.dot(v, w2_ref[...], preferred_element_type=jnp.float32)
    h2 = jnp.maximum(h2 + b2_ref[...], 0.0)
    part = jnp.sum(h2, axis=0, keepdims=True)

    @pl.when(i == 0)
    def _():
        acc_ref[...] = jnp.zeros_like(acc_ref)

    acc_ref[...] += part

    @pl.when(i == NB_TC - 1)
    def _():
        m = acc_ref[...] * (1.0 / N)
        out_ref[...] = (jnp.sum(m * wlt_ref[...], axis=1, keepdims=True)
                        + bl_ref[...])


def _tc3(agg2, u, dinv, W2, b2, Wlt, bl):
    return pl.pallas_call(
        _tc3_body,
        grid=(NB_TC,),
        in_specs=[
            pl.BlockSpec((4, BLK, 16), lambda i: (0, i, 0)),
            pl.BlockSpec((BLK, 64), lambda i: (i, 0)),
            pl.BlockSpec((BLK, 1), lambda i: (i, 0)),
            pl.BlockSpec((64, 64), lambda i: (0, 0)),
            pl.BlockSpec((1, 64), lambda i: (0, 0)),
            pl.BlockSpec((1, 64), lambda i: (0, 0)),
            pl.BlockSpec((1, 1), lambda i: (0, 0)),
        ],
        out_specs=pl.BlockSpec((1, 1), lambda i: (0, 0)),
        out_shape=jax.ShapeDtypeStruct((1, 1), jnp.float32),
        scratch_shapes=[pltpu.VMEM((1, 64), jnp.float32)],
    )(agg2, u, dinv, W2, b2, Wlt, bl)


@jax.jit
def kernel(x, edge_index, W1, b1, W2, b2, Wl, bl):
    src = edge_index[0].astype(jnp.int32)
    dst = edge_index[1].astype(jnp.int32)
    pad = E_PAD - E
    srcp = jnp.concatenate([src, jnp.zeros((pad,), jnp.int32)])
    dstp = jnp.concatenate([dst, jnp.full((pad,), TRASH, jnp.int32)])
    src3d = srcp.reshape(NW, CPW, CHUNK)
    dst3d = dstp.reshape(NW, CPW, CHUNK)
    src3d4 = (srcp * 4).reshape(NW, CPW, CHUNK)

    xpad = jnp.pad(x, ((0, 0), (0, 14)))
    z16 = jnp.zeros((N_PAD, 16), jnp.float32)
    ones_rows = jnp.zeros((CHUNK, 16), jnp.float32).at[:, 0].set(1.0)

    deg_parts = _sc_deg(dst3d, ones_rows, z16)
    dinv, g = _tc1(deg_parts, xpad)

    agg1 = _sc_agg(src3d, dst3d, g, z16, per_core_features=False)
    u = _tc2(agg1, g, dinv, W1, b1.reshape(1, 64))

    u4 = u.reshape(4 * N, 16)
    agg2 = _sc_agg(src3d4, dst3d, u4, z16, per_core_features=True)

    out = _tc3(agg2, u, dinv, W2, b2.reshape(1, 64), Wl.reshape(1, 64),
               bl.reshape(1, 1))
    return out.reshape(1)


# speedup vs baseline: 74.8790x; 1.0702x over previous
import functools

import jax
import jax.numpy as jnp
from jax import lax
from jax.experimental import pallas as pl
from jax.experimental.pallas import tpu as pltpu
from jax.experimental.pallas import tpu_sc as plsc

N = 50000
E = 3200000
NCORE = 2
NSUB = 16
NW = NCORE * NSUB
CHUNK = 128
BLKCH = 56
NBLK = 14
CPW = BLKCH * NBLK
E_PAD = NW * CPW * CHUNK
GRP = 7
N_PAD = 50048
STRIPE = N_PAD // NSUB
TRASH = N

BLK = 2000
NB_TC = N // BLK

_mesh = plsc.VectorSubcoreMesh(core_axis_name="c", subcore_axis_name="s")
_sc_params = pltpu.CompilerParams(use_tc_tiling_on_sc=False)


def _zero_stripe(z_hbm, acc, s):
    sl = pl.ds(s * STRIPE, STRIPE)
    pltpu.sync_copy(z_hbm.at[sl], acc.at[sl])


def _writeback_stripe(acc, out_hbm, c, s):
    sl = pl.ds(s * STRIPE, STRIPE)
    pltpu.sync_copy(acc.at[sl], out_hbm.at[c, sl])


def _scatter_const_row(dst3d_hbm, ones_hbm, z_hbm, out_hbm, dstb, ones_v, acc,
                       sem_s, sem_i):
    c = lax.axis_index("c")
    s = lax.axis_index("s")
    w = 2 * s + c
    _zero_stripe(z_hbm, acc, s)
    pltpu.async_copy(ones_hbm, ones_v, sem_i).wait()
    plsc.subcore_barrier()

    @pl.loop(0, NBLK)
    def _blk(b):
        pltpu.async_copy(dst3d_hbm.at[w, pl.ds(b * BLKCH, BLKCH)], dstb,
                         sem_i).wait()

        @pl.loop(0, BLKCH)
        def _ch(j):
            pltpu.async_copy(ones_v, acc.at[dstb.at[j]], sem_s, add=True)

        @pl.loop(0, BLKCH)
        def _dr(j):
            pltpu.make_async_copy(ones_v, acc.at[dstb.at[0]], sem_s).wait()

    plsc.subcore_barrier()
    _writeback_stripe(acc, out_hbm, c, s)


def _make_agg_kernel(width, per_core_features):
    def inner(src_hbm, dst3d_hbm, table_hbm, acc, srcb, dstb, rows,
              sem_g, sem_s, sem_i, w, qq):
        @pl.loop(0, NBLK)
        def _blk(b):
            pltpu.async_copy(src_hbm.at[w, pl.ds(b * BLKCH, BLKCH)], srcb,
                             sem_i).wait()
            pltpu.async_copy(dst3d_hbm.at[w, pl.ds(b * BLKCH, BLKCH)],
                             dstb, sem_i).wait()
            if per_core_features:
                @pl.loop(0, BLKCH)
                def _fix(r):
                    for k in range(CHUNK // 16):
                        sl = pl.ds(k * 16, 16)
                        srcb[r, sl] = srcb[r, sl] + qq

            @pl.loop(0, BLKCH // GRP)
            def _grp(gg):
                for p in range(GRP):
                    j = gg * GRP + p

                    @pl.when(gg > 0)
                    def _():
                        pltpu.make_async_copy(
                            rows.at[p], acc.at[dstb.at[0]],
                            sem_s.at[p]).wait()

                    pltpu.async_copy(table_hbm.at[srcb.at[j]], rows.at[p],
                                     sem_g.at[p])
                for p in range(GRP):
                    j = gg * GRP + p
                    pltpu.make_async_copy(table_hbm.at[srcb.at[j]],
                                          rows.at[p], sem_g.at[p]).wait()
                    pltpu.async_copy(rows.at[p], acc.at[dstb.at[j]],
                                     sem_s.at[p], add=True)

            for p in range(GRP):
                pltpu.make_async_copy(rows.at[p], acc.at[dstb.at[0]],
                                      sem_s.at[p]).wait()

    def body(src_hbm, dst3d_hbm, table_hbm, z_hbm, out_hbm,
             srcb, dstb, rows, acc, sem_g, sem_s, sem_i):
        c = lax.axis_index("c")
        s = lax.axis_index("s")

        if per_core_features:
            for q in range(2):
                qq = 2 * c + q
                _zero_stripe(z_hbm, acc, s)
                plsc.subcore_barrier()
                for rr in range(2):
                    inner(src_hbm, dst3d_hbm, table_hbm, acc, srcb, dstb,
                          rows, sem_g, sem_s, sem_i, 2 * s + rr, qq)
                plsc.subcore_barrier()
                sl = pl.ds(s * STRIPE, STRIPE)
                pltpu.sync_copy(acc.at[sl], out_hbm.at[qq, sl])
                plsc.subcore_barrier()
        else:
            _zero_stripe(z_hbm, acc, s)
            plsc.subcore_barrier()
            inner(src_hbm, dst3d_hbm, table_hbm, acc, srcb, dstb, rows,
                  sem_g, sem_s, sem_i, 2 * s + c, 0)
            plsc.subcore_barrier()
            _writeback_stripe(acc, out_hbm, c, s)

    return body


def _sc_deg(dst3d, ones_rows, z16):
    kern = functools.partial(
        pl.kernel,
        out_type=jax.ShapeDtypeStruct((NCORE, N_PAD, 16), jnp.float32),
        mesh=_mesh,
        compiler_params=_sc_params,
        scratch_types=[
            pltpu.VMEM((BLKCH, CHUNK), jnp.int32),
            pltpu.VMEM((CHUNK, 16), jnp.float32),
            pltpu.VMEM_SHARED((N_PAD, 16), jnp.float32),
            pltpu.SemaphoreType.DMA,
            pltpu.SemaphoreType.DMA,
        ],
    )(_scatter_const_row)
    return kern(dst3d, ones_rows, z16)


def _sc_agg(src_idx, dst3d, table, zeros, per_core_features):
    body = _make_agg_kernel(16, per_core_features)
    nout = 4 if per_core_features else NCORE
    kern = functools.partial(
        pl.kernel,
        out_type=jax.ShapeDtypeStruct((nout, N_PAD, 16), jnp.float32),
        mesh=_mesh,
        compiler_params=_sc_params,
        scratch_types=[
            pltpu.VMEM((BLKCH, CHUNK), jnp.int32),
            pltpu.VMEM((BLKCH, CHUNK), jnp.int32),
            pltpu.VMEM((GRP, CHUNK, 16), jnp.float32),
            pltpu.VMEM_SHARED((N_PAD, 16), jnp.float32),
            pltpu.SemaphoreType.DMA((GRP,)),
            pltpu.SemaphoreType.DMA((GRP,)),
            pltpu.SemaphoreType.DMA,
        ],
    )(body)
    return kern(src_idx, dst3d, table, zeros)


def _tc1_body(parts_ref, xpad_ref, dinv_ref, g_ref):
    deg = parts_ref[0, :, 0:1] + parts_ref[1, :, 0:1] + 1.0
    dv = lax.rsqrt(deg)
    dinv_ref[...] = dv
    g_ref[...] = xpad_ref[...] * dv


def _tc1(parts, xpad):
    return pl.pallas_call(
        _tc1_body,
        grid=(NB_TC,),
        in_specs=[
            pl.BlockSpec((NCORE, BLK, 16), lambda i: (0, i, 0)),
            pl.BlockSpec((BLK, 16), lambda i: (i, 0)),
        ],
        out_specs=[
            pl.BlockSpec((BLK, 1), lambda i: (i, 0)),
            pl.BlockSpec((BLK, 16), lambda i: (i, 0)),
        ],
        out_shape=[
            jax.ShapeDtypeStruct((N, 1), jnp.float32),
            jax.ShapeDtypeStruct((N, 16), jnp.float32),
        ],
    )(parts, xpad)


def _tc2_body(agg_ref, g_ref, dinv_ref, w1_ref, b1_ref, u_ref):
    a = agg_ref[0] + agg_ref[1] + g_ref[...]
    dv = dinv_ref[...]
    y = dv * a[:, 0:2]
    h = (y[:, 0:1] * w1_ref[0:1, :] + y[:, 1:2] * w1_ref[1:2, :]
         + b1_ref[...])
    h = jnp.maximum(h, 0.0)
    u_ref[...] = dv * h


def _tc2(agg1, g, dinv, W1, b1):
    return pl.pallas_call(
        _tc2_body,
        grid=(NB_TC,),
        in_specs=[
            pl.BlockSpec((NCORE, BLK, 16), lambda i: (0, i, 0)),
            pl.BlockSpec((BLK, 16), lambda i: (i, 0)),
            pl.BlockSpec((BLK, 1), lambda i: (i, 0)),
            pl.BlockSpec((2, 64), lambda i: (0, 0)),
            pl.BlockSpec((1, 64), lambda i: (0, 0)),
        ],
        out_specs=pl.BlockSpec((BLK, 64), lambda i: (i, 0)),
        out_shape=jax.ShapeDtypeStruct((N, 64), jnp.float32),
    )(agg1, g, dinv, W1, b1)


def _tc3_body(agg_ref, u_ref, dinv_ref, w2_ref, b2_ref, wlt_ref, bl_ref,
              out_ref, acc_ref):
    i = pl.program_id(0)
    dv = dinv_ref[...]
    agg = jnp.concatenate([agg_ref[0], agg_ref[1], agg_ref[2], agg_ref[3]],
                          axis=1)
    v = dv * (agg + u_ref[...])
    h2 = jnp.dot(v, w2_ref[...], preferred_element_type=jnp.float32)
    h2 = jnp.maximum(h2 + b2_ref[...], 0.0)
    part = jnp.sum(h2, axis=0, keepdims=True)

    @pl.when(i == 0)
    def _():
        acc_ref[...] = jnp.zeros_like(acc_ref)

    acc_ref[...] += part

    @pl.when(i == NB_TC - 1)
    def _():
        m = acc_ref[...] * (1.0 / N)
        out_ref[...] = (jnp.sum(m * wlt_ref[...], axis=1, keepdims=True)
                        + bl_ref[...])


def _tc3(agg2, u, dinv, W2, b2, Wlt, bl):
    return pl.pallas_call(
        _tc3_body,
        grid=(NB_TC,),
        in_specs=[
            pl.BlockSpec((4, BLK, 16), lambda i: (0, i, 0)),
            pl.BlockSpec((BLK, 64), lambda i: (i, 0)),
            pl.BlockSpec((BLK, 1), lambda i: (i, 0)),
            pl.BlockSpec((64, 64), lambda i: (0, 0)),
            pl.BlockSpec((1, 64), lambda i: (0, 0)),
            pl.BlockSpec((1, 64), lambda i: (0, 0)),
            pl.BlockSpec((1, 1), lambda i: (0, 0)),
        ],
        out_specs=pl.BlockSpec((1, 1), lambda i: (0, 0)),
        out_shape=jax.ShapeDtypeStruct((1, 1), jnp.float32),
        scratch_shapes=[pltpu.VMEM((1, 64), jnp.float32)],
    )(agg2, u, dinv, W2, b2, Wlt, bl)


@jax.jit
def kernel(x, edge_index, W1, b1, W2, b2, Wl, bl):
    src = edge_index[0].astype(jnp.int32)
    dst = edge_index[1].astype(jnp.int32)
    pad = E_PAD - E
    srcp = jnp.concatenate([src, jnp.zeros((pad,), jnp.int32)])
    dstp = jnp.concatenate([dst, jnp.full((pad,), TRASH, jnp.int32)])
    src3d = srcp.reshape(NW, CPW, CHUNK)
    dst3d = dstp.reshape(NW, CPW, CHUNK)
    src3d4 = (srcp * 4).reshape(NW, CPW, CHUNK)

    xpad = jnp.pad(x, ((0, 0), (0, 14)))
    z16 = jnp.zeros((N_PAD, 16), jnp.float32)
    ones_rows = jnp.zeros((CHUNK, 16), jnp.float32).at[:, 0].set(1.0)

    deg_parts = _sc_deg(dst3d, ones_rows, z16)
    dinv, g = _tc1(deg_parts, xpad)

    agg1 = _sc_agg(src3d, dst3d, g, z16, per_core_features=False)
    u = _tc2(agg1, g, dinv, W1, b1.reshape(1, 64))

    u4 = u.reshape(4 * N, 16)
    agg2 = _sc_agg(src3d4, dst3d, u4, z16, per_core_features=True)

    out = _tc3(agg2, u, dinv, W2, b2.reshape(1, 64), Wl.reshape(1, 64),
               bl.reshape(1, 1))
    return out.reshape(1)
